# Initial kernel scaffold; baseline (speedup 1.0000x reference)
import jax
import jax.numpy as jnp
from jax.experimental import pallas as pl


def kernel(inputs, table):
    raise NotImplementedError("write your pallas kernel here")


# speedup vs baseline: 2.7081x; 2.7081x over previous
import functools

import jax
import jax.numpy as jnp
from jax import lax
from jax.experimental import pallas as pl
from jax.experimental.pallas import tpu as pltpu
from jax.experimental.pallas import tpu_sc as plsc

NUM_CORES = 2
NUM_SUBCORES = 16
NUM_WORKERS = NUM_CORES * NUM_SUBCORES

IDX_PER_STREAM = 128
STREAMS_PER_CHUNK = 8
CHUNK = IDX_PER_STREAM * STREAMS_PER_CHUNK


def _sc_embed(flat_idx2d, table, batch, dim):
    per_worker = batch // NUM_WORKERS
    chunks_per_worker = per_worker // CHUNK
    idx_rows_per_chunk = CHUNK // IDX_PER_STREAM

    mesh = plsc.VectorSubcoreMesh(core_axis_name="c", subcore_axis_name="s")

    @functools.partial(
        pl.kernel,
        out_type=jax.ShapeDtypeStruct((batch, dim), jnp.float32),
        mesh=mesh,
        scratch_types=[
            pltpu.VMEM((idx_rows_per_chunk, IDX_PER_STREAM), jnp.int32),
            pltpu.VMEM((CHUNK, dim), jnp.float32),
            pltpu.SemaphoreType.DMA,
        ],
        compiler_params=pltpu.CompilerParams(use_tc_tiling_on_sc=False),
    )
    def k(table_hbm, idx_hbm, out_hbm, idx_v, rows_v, sem):
        wid = lax.axis_index("s") * NUM_CORES + lax.axis_index("c")
        base_row = wid * (per_worker // IDX_PER_STREAM)

        def body(g, carry):
            pltpu.sync_copy(
                idx_hbm.at[pl.ds(base_row + g * idx_rows_per_chunk,
                                 idx_rows_per_chunk)],
                idx_v,
            )
            copies = []
            for j in range(STREAMS_PER_CHUNK):
                copies.append(pltpu.async_copy(
                    table_hbm.at[idx_v.at[j]],
                    rows_v.at[pl.ds(j * IDX_PER_STREAM, IDX_PER_STREAM)],
                    sem,
                ))
            for c in copies:
                c.wait()
            pltpu.sync_copy(
                rows_v,
                out_hbm.at[pl.ds(wid * per_worker + g * CHUNK, CHUNK)],
            )
            return carry

        lax.fori_loop(0, chunks_per_worker, body, 0)

    return k(table, flat_idx2d)


def kernel(inputs, table):
    b, t = inputs.shape
    vocab, dim = table.shape
    batch = b * t
    flat_idx2d = inputs.reshape(batch // IDX_PER_STREAM,
                                IDX_PER_STREAM).astype(jnp.int32)
    out = _sc_embed(flat_idx2d, table, batch, dim)
    return out.reshape(b, t, dim)


# speedup vs baseline: 4.6943x; 1.7335x over previous
import functools

import jax
import jax.numpy as jnp
from jax import lax
from jax.experimental import pallas as pl
from jax.experimental.pallas import tpu as pltpu
from jax.experimental.pallas import tpu_sc as plsc

NUM_CORES = 2
NUM_SUBCORES = 16
NUM_WORKERS = NUM_CORES * NUM_SUBCORES

IDX_PER_STREAM = 128
STREAMS_PER_CHUNK = 5
CHUNK = IDX_PER_STREAM * STREAMS_PER_CHUNK


def _sc_embed(flat_idx2d, table, batch, dim):
    vocab = table.shape[0]
    per_worker = batch // NUM_WORKERS
    idx_rows_per_worker = per_worker // IDX_PER_STREAM
    n_chunks = per_worker // CHUNK
    assert n_chunks % 2 == 0

    mesh = plsc.VectorSubcoreMesh(core_axis_name="c", subcore_axis_name="s")

    @functools.partial(
        pl.kernel,
        out_type=jax.ShapeDtypeStruct((batch, dim), jnp.float32),
        mesh=mesh,
        scratch_types=[
            pltpu.VMEM_SHARED((vocab, dim), jnp.float32),
            pltpu.VMEM((idx_rows_per_worker, IDX_PER_STREAM), jnp.int32),
            pltpu.VMEM((CHUNK, dim), jnp.float32),
            pltpu.VMEM((CHUNK, dim), jnp.float32),
            pltpu.SemaphoreType.DMA,
            pltpu.SemaphoreType.DMA,
            pltpu.SemaphoreType.DMA,
            pltpu.SemaphoreType.DMA,
        ],
        compiler_params=pltpu.CompilerParams(use_tc_tiling_on_sc=False),
    )
    def k(table_hbm, idx_hbm, out_hbm, table_v, idx_v, rows0, rows1,
          g0sem, g1sem, o0sem, o1sem):
        wid = lax.axis_index("s") * NUM_CORES + lax.axis_index("c")
        out_base = wid * per_worker

        @pl.when(lax.axis_index("s") == 0)
        def _():
            pltpu.sync_copy(table_hbm, table_v)

        plsc.subcore_barrier()
        pltpu.sync_copy(
            idx_hbm.at[pl.ds(wid * idx_rows_per_worker, idx_rows_per_worker)],
            idx_v,
        )

        def fire_gather(c, rows, sem):
            for j in range(STREAMS_PER_CHUNK):
                pltpu.async_copy(
                    table_v.at[idx_v.at[c * STREAMS_PER_CHUNK + j]],
                    rows.at[pl.ds(j * IDX_PER_STREAM, IDX_PER_STREAM)],
                    sem,
                )

        def wait_gather(rows, sem):
            pltpu.make_async_copy(out_hbm.at[pl.ds(0, CHUNK)], rows, sem).wait()

        def fire_out(c, rows, sem):
            pltpu.async_copy(
                rows, out_hbm.at[pl.ds(out_base + c * CHUNK, CHUNK)], sem)

        def wait_out(rows, sem):
            pltpu.make_async_copy(rows, out_hbm.at[pl.ds(0, CHUNK)], sem).wait()

        fire_gather(0, rows0, g0sem)
        fire_gather(1, rows1, g1sem)

        def body(i, carry):
            c0 = 2 * i
            c1 = c0 + 1
            wait_gather(rows0, g0sem)
            fire_out(c0, rows0, o0sem)
            wait_gather(rows1, g1sem)
            fire_out(c1, rows1, o1sem)
            wait_out(rows0, o0sem)
            fire_gather(c0 + 2, rows0, g0sem)
            wait_out(rows1, o1sem)
            fire_gather(c1 + 2, rows1, g1sem)
            return carry

        lax.fori_loop(0, n_chunks // 2 - 1, body, 0)

        c0 = n_chunks - 2
        wait_gather(rows0, g0sem)
        fire_out(c0, rows0, o0sem)
        wait_gather(rows1, g1sem)
        fire_out(c0 + 1, rows1, o1sem)
        wait_out(rows0, o0sem)
        wait_out(rows1, o1sem)

    return k(table, flat_idx2d)


def kernel(inputs, table):
    b, t = inputs.shape
    vocab, dim = table.shape
    batch = b * t
    flat_idx2d = inputs.reshape(batch // IDX_PER_STREAM,
                                IDX_PER_STREAM).astype(jnp.int32)
    out = _sc_embed(flat_idx2d, table, batch, dim)
    return out.reshape(b, t, dim)
